# baseline (speedup 1.0000x reference)
import jax
import jax.numpy as jnp
from jax import lax
from jax.experimental import pallas as pl
from jax.experimental.pallas import tpu as pltpu
from jax.experimental.pallas import tpu_sc as plsc

NC = 2
NS = 16
NW = NC * NS

NBUF = 4
LANES = 16


def _make_sc_kernel(total, D, seq, scale):
    chunk = seq
    subg = seq // 2
    gpc = chunk // subg
    per_w = total // NW
    nch = per_w // chunk
    nvec = D // LANES

    mesh = plsc.VectorSubcoreMesh(core_axis_name="c", subcore_axis_name="s")

    def body(x_hbm, table_hbm, pe_hbm, out_hbm, idx_v, buf, pe_v, gsem, ssem):
        ci = lax.axis_index("c")
        si = lax.axis_index("s")
        w = si * NC + ci
        base = w * per_w

        pltpu.sync_copy(x_hbm.at[w], idx_v)
        pltpu.sync_copy(pe_hbm.at[pl.ds(0, seq)], pe_v)

        def start_gather(c, b):
            for i in range(gpc):
                pltpu.async_copy(
                    table_hbm.at[idx_v.at[c * gpc + i]],
                    buf.at[b, pl.ds(i * subg, subg)],
                    gsem.at[b],
                )

        def wait_gather(b):
            pltpu.make_async_copy(
                table_hbm.at[pl.ds(0, chunk)], buf.at[b], gsem.at[b]
            ).wait()

        def start_scatter(c, b):
            pltpu.async_copy(
                buf.at[b], out_hbm.at[pl.ds(base + c * chunk, chunk)], ssem.at[b]
            )

        def wait_scatter(b):
            pltpu.make_async_copy(
                buf.at[b], out_hbm.at[pl.ds(0, chunk)], ssem.at[b]
            ).wait()

        start_gather(0, 0)
        start_gather(1, 1)

        def chunk_body(c, b):
            b2 = (b + 2) % NBUF

            @pl.when(c >= 2)
            def _():
                wait_scatter(b2)

            @pl.when(c + 2 < nch)
            def _():
                start_gather(c + 2, b2)

            wait_gather(b)

            def row_body(r, carry):
                for j in range(nvec):
                    sl = pl.ds(j * LANES, LANES)
                    buf[b, r, sl] = buf[b, r, sl] * scale + pe_v[r, sl]
                return carry

            lax.fori_loop(0, chunk, row_body, 0)
            start_scatter(c, b)

        def outer(t, carry):
            c0 = t * NBUF
            for b in range(NBUF):
                chunk_body(c0 + b, b)
            return carry

        lax.fori_loop(0, nch // NBUF, outer, 0)

        wait_scatter((nch - 2) % NBUF)
        wait_scatter((nch - 1) % NBUF)

    return pl.kernel(
        body,
        out_type=jax.ShapeDtypeStruct((total, D), jnp.float32),
        mesh=mesh,
        scratch_types=[
            pltpu.VMEM((per_w // subg, subg), jnp.int32),
            pltpu.VMEM((NBUF, chunk, D), jnp.float32),
            pltpu.VMEM((seq, D), jnp.float32),
            pltpu.SemaphoreType.DMA((NBUF,)),
            pltpu.SemaphoreType.DMA((NBUF,)),
        ],
        compiler_params=pltpu.CompilerParams(use_tc_tiling_on_sc=False),
    )


def kernel(x, table, pos_encoding):
    B, S = x.shape
    V, D = table.shape
    scale = float(D) ** 0.5
    total = B * S
    per_w = total // NW
    subg = S // 2
    assert total % NW == 0 and S % 2 == 0 and subg <= 128
    assert per_w % S == 0 and (per_w // S) % NBUF == 0 and D % LANES == 0

    xf = x.astype(jnp.int32).reshape(NW, per_w // subg, subg)
    out = _make_sc_kernel(total, D, S, scale)(xf, table, pos_encoding)
    return out.reshape(B, S, D)


# speedup vs baseline: 1.0097x; 1.0097x over previous
import jax
import jax.numpy as jnp
from jax import lax
from jax.experimental import pallas as pl
from jax.experimental.pallas import tpu as pltpu
from jax.experimental.pallas import tpu_sc as plsc

NC = 2
NS = 16
NW = NC * NS

NBUF = 4
LANES = 16


def _make_sc_kernel(total, D, seq, scale):
    chunk = seq
    subg = seq // 2
    gpc = chunk // subg
    per_w = total // NW
    nch = per_w // chunk
    nvec = D // LANES

    mesh = plsc.VectorSubcoreMesh(core_axis_name="c", subcore_axis_name="s")

    def body(x_hbm, table_hbm, pe_hbm, out_hbm, idx_v, buf, pe_v, gsem, ssem):
        ci = lax.axis_index("c")
        si = lax.axis_index("s")
        w = si * NC + ci
        base = w * per_w

        pltpu.sync_copy(x_hbm.at[w], idx_v)
        pltpu.sync_copy(pe_hbm.at[pl.ds(0, seq)], pe_v)

        def start_gather(c, b):
            for i in range(gpc):
                pltpu.async_copy(
                    table_hbm.at[idx_v.at[c * gpc + i]],
                    buf.at[b, pl.ds(i * subg, subg)],
                    gsem.at[b],
                )

        def wait_gather(b):
            pltpu.make_async_copy(
                table_hbm.at[pl.ds(0, chunk)], buf.at[b], gsem.at[b]
            ).wait()

        def start_scatter(c, b):
            pltpu.async_copy(
                buf.at[b], out_hbm.at[pl.ds(base + c * chunk, chunk)], ssem.at[b]
            )

        def wait_scatter(b):
            pltpu.make_async_copy(
                buf.at[b], out_hbm.at[pl.ds(0, chunk)], ssem.at[b]
            ).wait()

        start_gather(0, 0)
        start_gather(1, 1)

        def chunk_body(c, b):
            b2 = (b + 2) % NBUF

            @pl.when(c >= 2)
            def _():
                wait_scatter(b2)

            @pl.when(c + 2 < nch)
            def _():
                start_gather(c + 2, b2)

            wait_gather(b)

            @plsc.parallel_loop(0, chunk, step=1, unroll=8)
            def _(r):
                for j in range(nvec):
                    sl = pl.ds(j * LANES, LANES)
                    buf[b, r, sl] = buf[b, r, sl] * scale + pe_v[r, sl]

            start_scatter(c, b)

        def outer(t, carry):
            c0 = t * NBUF
            for b in range(NBUF):
                chunk_body(c0 + b, b)
            return carry

        lax.fori_loop(0, nch // NBUF, outer, 0)

        wait_scatter((nch - 2) % NBUF)
        wait_scatter((nch - 1) % NBUF)

    return pl.kernel(
        body,
        out_type=jax.ShapeDtypeStruct((total, D), jnp.float32),
        mesh=mesh,
        scratch_types=[
            pltpu.VMEM((per_w // subg, subg), jnp.int32),
            pltpu.VMEM((NBUF, chunk, D), jnp.float32),
            pltpu.VMEM((seq, D), jnp.float32),
            pltpu.SemaphoreType.DMA((NBUF,)),
            pltpu.SemaphoreType.DMA((NBUF,)),
        ],
        compiler_params=pltpu.CompilerParams(use_tc_tiling_on_sc=False),
    )


def kernel(x, table, pos_encoding):
    B, S = x.shape
    V, D = table.shape
    scale = float(D) ** 0.5
    total = B * S
    per_w = total // NW
    subg = S // 2
    assert total % NW == 0 and S % 2 == 0 and subg <= 128
    assert per_w % S == 0 and (per_w // S) % NBUF == 0 and D % LANES == 0

    xf = x.astype(jnp.int32).reshape(NW, per_w // subg, subg)
    out = _make_sc_kernel(total, D, S, scale)(xf, table, pos_encoding)
    return out.reshape(B, S, D)
